# Initial kernel scaffold; baseline (speedup 1.0000x reference)
import jax
import jax.numpy as jnp
from jax.experimental import pallas as pl


def kernel(x, edge_index, edge_vals, W, b):
    raise NotImplementedError("write your pallas kernel here")


# speedup vs baseline: 7.7313x; 7.7313x over previous
import functools

import jax
import jax.numpy as jnp
from jax import lax
from jax.experimental import pallas as pl
from jax.experimental.pallas import tpu as pltpu
from jax.experimental.pallas import tpu_sc as plsc

_NC = 2
_NS = 16
_NW = _NC * _NS
_L = 16

_C = 48
_NBUF = 4
_BR = 40


def _sc_gather_scatter(x, colp, rowp, evp, n, d, nchunk):
    nrounds = nchunk // _NBUF
    nblk = n // _BR

    mesh = plsc.VectorSubcoreMesh(core_axis_name="c", subcore_axis_name="s")

    @functools.partial(
        pl.kernel,
        out_type=jax.ShapeDtypeStruct((_NC, n, d), jnp.float32),
        mesh=mesh,
        compiler_params=pltpu.CompilerParams(needs_layout_passes=False),
        scratch_types=[
            pltpu.VMEM((_NBUF, _C), jnp.int32),
            pltpu.VMEM((_NBUF, _C), jnp.int32),
            pltpu.VMEM((_NBUF * _C,), jnp.float32),
            pltpu.VMEM((_NBUF, _C, d), jnp.float32),
            pltpu.VMEM_SHARED((n, d), jnp.float32),
            pltpu.SemaphoreType.DMA((_NBUF,)),
            pltpu.SemaphoreType.DMA((_NBUF,)),
        ],
    )
    def sc_kernel(x_hbm, col_hbm, row_hbm, ev_hbm, out_hbm,
                  colb, rowb, evb, msgs_v, agg_s, isem, gsem):
        c = lax.axis_index("c")
        s = lax.axis_index("s")
        w = s * _NC + c

        def zero_body(e, carry):
            for v in range(d // _L):
                msgs_v[0, e, pl.ds(v * _L, _L)] = jnp.zeros((_L,), jnp.float32)
            return carry
        lax.fori_loop(0, _BR, zero_body, 0)
        for k in range(-(-nblk // _NS)):
            blk = k * _NS + s

            @pl.when(blk < nblk)
            def _():
                pltpu.sync_copy(msgs_v.at[0, pl.ds(0, _BR)],
                                agg_s.at[pl.ds(blk * _BR, _BR)])
        plsc.subcore_barrier()

        def idx_copies(chunk, buf):
            base = (w * nchunk + chunk) * _C
            return (
                pltpu.make_async_copy(col_hbm.at[pl.ds(base, _C)],
                                      colb.at[buf], isem.at[buf]),
                pltpu.make_async_copy(row_hbm.at[pl.ds(base, _C)],
                                      rowb.at[buf], isem.at[buf]),
                pltpu.make_async_copy(ev_hbm.at[pl.ds(base, _C)],
                                      evb.at[pl.ds(buf * _C, _C)],
                                      isem.at[buf]),
            )

        def start_idx(chunk, buf):
            for cp in idx_copies(chunk, buf):
                cp.start()

        def wait_idx(chunk, buf):
            for cp in idx_copies(chunk, buf):
                cp.wait()

        def start_gather(buf):
            pltpu.async_copy(x_hbm.at[colb.at[buf]], msgs_v.at[buf],
                             gsem.at[buf])

        def wait_gather(buf):
            pltpu.make_async_copy(x_hbm.at[colb.at[buf]], msgs_v.at[buf],
                                  gsem.at[buf]).wait()

        def scale(buf):
            def body(e, carry):
                idx = jnp.full((_L,), buf * _C, jnp.int32) + e
                evs = plsc.load_gather(evb, [idx])
                for v in range(d // _L):
                    cur = msgs_v[buf, e, pl.ds(v * _L, _L)]
                    msgs_v[buf, e, pl.ds(v * _L, _L)] = cur * evs
                return carry
            lax.fori_loop(0, _C, body, 0)

        start_idx(0, 0)
        start_idx(1, 1)
        wait_idx(0, 0)
        start_gather(0)

        def round_body(r, carry):
            for b in range(_NBUF):
                i = r * _NBUF + b

                @pl.when(i + 2 < nchunk)
                def _():
                    start_idx(i + 2, (b + 2) % _NBUF)

                @pl.when(i + 1 < nchunk)
                def _():
                    wait_idx(i + 1, (b + 1) % _NBUF)
                    start_gather((b + 1) % _NBUF)

                wait_gather(b)
                scale(b)
                pltpu.sync_copy(msgs_v.at[b], agg_s.at[rowb.at[b]], add=True)
            return carry

        lax.fori_loop(0, nrounds, round_body, 0)

        plsc.subcore_barrier()
        for k in range(-(-nblk // _NS)):
            blk = k * _NS + s

            @pl.when(blk < nblk)
            def _():
                pltpu.sync_copy(agg_s.at[pl.ds(blk * _BR, _BR)],
                                out_hbm.at[c, pl.ds(blk * _BR, _BR)])

    return sc_kernel(x, colp, rowp, evp)


def _tc_combine_matmul(partial, W, b2, n, d):
    mb = 2000

    def body(p_ref, w_ref, b_ref, o_ref):
        agg = p_ref[0] + p_ref[1]
        o_ref[...] = (
            jnp.dot(agg, w_ref[...], preferred_element_type=jnp.float32)
            + b_ref[...]
        )

    return pl.pallas_call(
        body,
        grid=(n // mb,),
        in_specs=[
            pl.BlockSpec((_NC, mb, d), lambda i: (0, i, 0)),
            pl.BlockSpec((d, d), lambda i: (0, 0)),
            pl.BlockSpec((1, d), lambda i: (0, 0)),
        ],
        out_specs=pl.BlockSpec((mb, d), lambda i: (i, 0)),
        out_shape=jax.ShapeDtypeStruct((n, d), jnp.float32),
    )(partial, W, b2)


def kernel(x, edge_index, edge_vals, W, b):
    n, d_in = x.shape
    d_out = W.shape[1]
    e = edge_index.shape[1]

    ep = e // _NW
    nchunk = -(-ep // _C)
    nchunk = -(-nchunk // _NBUF) * _NBUF
    epp = nchunk * _C
    pad = epp - ep

    row = edge_index[0].reshape(_NW, ep)
    col = edge_index[1].reshape(_NW, ep)
    ev = edge_vals.reshape(_NW, ep)
    if pad:
        spread = (jnp.arange(_NW * pad, dtype=jnp.int32) % n).reshape(_NW, pad)
        row = jnp.concatenate([row, spread], axis=1)
        col = jnp.concatenate([col, spread], axis=1)
        ev = jnp.concatenate([ev, jnp.zeros((_NW, pad), jnp.float32)], axis=1)
    colp = col.reshape(_NW * epp)
    rowp = row.reshape(_NW * epp)
    evp = ev.reshape(_NW * epp)

    partial = _sc_gather_scatter(x, colp, rowp, evp, n, d_in, nchunk)
    return _tc_combine_matmul(partial, W, b.reshape(1, d_out), n, d_out)


# speedup vs baseline: 9.9846x; 1.2915x over previous
import functools

import jax
import jax.numpy as jnp
from jax import lax
from jax.experimental import pallas as pl
from jax.experimental.pallas import tpu as pltpu
from jax.experimental.pallas import tpu_sc as plsc

_NC = 2
_NS = 16
_NW = _NC * _NS
_L = 16

_C = 48
_NBUF = 4
_BR = 40


def _sc_gather_scatter(x, colp, rowp, evp, n, d, nchunk):
    nrounds = nchunk // _NBUF
    nblk = n // _BR

    mesh = plsc.VectorSubcoreMesh(core_axis_name="c", subcore_axis_name="s")

    @functools.partial(
        pl.kernel,
        out_type=jax.ShapeDtypeStruct((_NC, n, d), jnp.float32),
        mesh=mesh,
        compiler_params=pltpu.CompilerParams(needs_layout_passes=False),
        scratch_types=[
            pltpu.VMEM((_NBUF, _C), jnp.int32),
            pltpu.VMEM((_NBUF, _C), jnp.int32),
            pltpu.VMEM((_NBUF * _C,), jnp.float32),
            pltpu.VMEM((_NBUF, _C, d), jnp.float32),
            pltpu.VMEM_SHARED((n, d), jnp.float32),
            pltpu.SemaphoreType.DMA((_NBUF,)),
            pltpu.SemaphoreType.DMA((_NBUF,)),
            pltpu.SemaphoreType.DMA((_NBUF,)),
            pltpu.SemaphoreType.DMA,
        ],
    )
    def sc_kernel(x_hbm, col_hbm, row_hbm, ev_hbm, out_hbm,
                  colb, rowb, evb, msgs_v, agg_s, isem, gsem, ssem, bsem):
        c = lax.axis_index("c")
        s = lax.axis_index("s")
        w = s * _NC + c

        def zero_body(e, carry):
            for v in range(d // _L):
                msgs_v[0, e, pl.ds(v * _L, _L)] = jnp.zeros((_L,), jnp.float32)
            return carry
        lax.fori_loop(0, _BR, zero_body, 0)
        for k in range(-(-nblk // _NS)):
            blk = k * _NS + s

            @pl.when(blk < nblk)
            def _():
                pltpu.async_copy(msgs_v.at[0, pl.ds(0, _BR)],
                                 agg_s.at[pl.ds(blk * _BR, _BR)], bsem)
        for k in range(-(-nblk // _NS)):
            blk = k * _NS + s

            @pl.when(blk < nblk)
            def _():
                pltpu.make_async_copy(msgs_v.at[0, pl.ds(0, _BR)],
                                      agg_s.at[pl.ds(blk * _BR, _BR)],
                                      bsem).wait()
        plsc.subcore_barrier()

        def idx_copies(chunk, buf):
            base = (w * nchunk + chunk) * _C
            return (
                pltpu.make_async_copy(col_hbm.at[pl.ds(base, _C)],
                                      colb.at[buf], isem.at[buf]),
                pltpu.make_async_copy(row_hbm.at[pl.ds(base, _C)],
                                      rowb.at[buf], isem.at[buf]),
                pltpu.make_async_copy(ev_hbm.at[pl.ds(base, _C)],
                                      evb.at[pl.ds(buf * _C, _C)],
                                      isem.at[buf]),
            )

        def start_idx(chunk, buf):
            for cp in idx_copies(chunk, buf):
                cp.start()

        def wait_idx(chunk, buf):
            for cp in idx_copies(chunk, buf):
                cp.wait()

        def start_gather(buf):
            pltpu.async_copy(x_hbm.at[colb.at[buf]], msgs_v.at[buf],
                             gsem.at[buf])

        def wait_gather(buf):
            pltpu.make_async_copy(x_hbm.at[colb.at[buf]], msgs_v.at[buf],
                                  gsem.at[buf]).wait()

        def scale(buf):
            @plsc.parallel_loop(0, _C, unroll=4)
            def _(e):
                idx = jnp.full((_L,), buf * _C, jnp.int32) + e
                evs = plsc.load_gather(evb, [idx])
                for v in range(d // _L):
                    cur = msgs_v[buf, e, pl.ds(v * _L, _L)]
                    msgs_v[buf, e, pl.ds(v * _L, _L)] = cur * evs

        def start_scatter(buf):
            pltpu.async_copy(msgs_v.at[buf], agg_s.at[rowb.at[buf]],
                             ssem.at[buf], add=True)

        def wait_scatter(buf):
            pltpu.make_async_copy(msgs_v.at[buf], agg_s.at[rowb.at[buf]],
                                  ssem.at[buf]).wait()

        start_idx(0, 0)
        start_idx(1, 1)
        wait_idx(0, 0)
        start_gather(0)

        def round_body(r, carry):
            for b in range(_NBUF):
                i = r * _NBUF + b

                @pl.when(i >= 2)
                def _():
                    wait_scatter((b + 2) % _NBUF)

                @pl.when(i + 2 < nchunk)
                def _():
                    start_idx(i + 2, (b + 2) % _NBUF)

                @pl.when(i + 1 < nchunk)
                def _():
                    wait_idx(i + 1, (b + 1) % _NBUF)
                    start_gather((b + 1) % _NBUF)

                wait_gather(b)
                scale(b)
                start_scatter(b)
            return carry

        lax.fori_loop(0, nrounds, round_body, 0)
        wait_scatter((nchunk - 2) % _NBUF)
        wait_scatter((nchunk - 1) % _NBUF)

        plsc.subcore_barrier()
        for k in range(-(-nblk // _NS)):
            blk = k * _NS + s

            @pl.when(blk < nblk)
            def _():
                pltpu.async_copy(agg_s.at[pl.ds(blk * _BR, _BR)],
                                 out_hbm.at[c, pl.ds(blk * _BR, _BR)], bsem)
        for k in range(-(-nblk // _NS)):
            blk = k * _NS + s

            @pl.when(blk < nblk)
            def _():
                pltpu.make_async_copy(agg_s.at[pl.ds(blk * _BR, _BR)],
                                      out_hbm.at[c, pl.ds(blk * _BR, _BR)],
                                      bsem).wait()

    return sc_kernel(x, colp, rowp, evp)


def _tc_combine_matmul(partial, W, b2, n, d):
    mb = 2000

    def body(p_ref, w_ref, b_ref, o_ref):
        agg = p_ref[0] + p_ref[1]
        o_ref[...] = (
            jnp.dot(agg, w_ref[...], preferred_element_type=jnp.float32)
            + b_ref[...]
        )

    return pl.pallas_call(
        body,
        grid=(n // mb,),
        in_specs=[
            pl.BlockSpec((_NC, mb, d), lambda i: (0, i, 0)),
            pl.BlockSpec((d, d), lambda i: (0, 0)),
            pl.BlockSpec((1, d), lambda i: (0, 0)),
        ],
        out_specs=pl.BlockSpec((mb, d), lambda i: (i, 0)),
        out_shape=jax.ShapeDtypeStruct((n, d), jnp.float32),
    )(partial, W, b2)


def kernel(x, edge_index, edge_vals, W, b):
    n, d_in = x.shape
    d_out = W.shape[1]
    e = edge_index.shape[1]

    ep = e // _NW
    nchunk = -(-ep // _C)
    nchunk = -(-nchunk // _NBUF) * _NBUF
    epp = nchunk * _C
    pad = epp - ep

    row = edge_index[0].reshape(_NW, ep)
    col = edge_index[1].reshape(_NW, ep)
    ev = edge_vals.reshape(_NW, ep)
    if pad:
        spread = (jnp.arange(_NW * pad, dtype=jnp.int32) % n).reshape(_NW, pad)
        row = jnp.concatenate([row, spread], axis=1)
        col = jnp.concatenate([col, spread], axis=1)
        ev = jnp.concatenate([ev, jnp.zeros((_NW, pad), jnp.float32)], axis=1)
    colp = col.reshape(_NW * epp)
    rowp = row.reshape(_NW * epp)
    evp = ev.reshape(_NW * epp)

    partial = _sc_gather_scatter(x, colp, rowp, evp, n, d_in, nchunk)
    return _tc_combine_matmul(partial, W, b.reshape(1, d_out), n, d_out)
